# baseline (speedup 1.0000x reference)
import jax
import jax.numpy as jnp
from jax.experimental import pallas as pl
from jax.experimental.pallas import tpu as pltpu

_N = 20000
_R, _C = 160, 128
_K = 500
_KPAD = 512
_NCLS = 10


def _topk_body(cls_ref, box_ref, scores_out, boxes_out, labels_out, s_scr, l_scr):
    x = cls_ref[...]
    best = x[0]
    lab = jnp.zeros((_R, _C), jnp.int32)
    for c in range(1, _NCLS):
        upd = x[c] > best
        best = jnp.where(upd, x[c], best)
        lab = jnp.where(upd, c, lab)

    ir = jax.lax.broadcasted_iota(jnp.int32, (_R, _C), 0)
    ic = jax.lax.broadcasted_iota(jnp.int32, (_R, _C), 1)
    flat = ir * _C + ic
    best = jnp.where(flat < _N, best, -jnp.inf)
    s_scr[...] = best
    l_scr[...] = lab + 1

    def body(i, _):
        s = s_scr[...]
        m = jnp.max(s)
        cand = jnp.where(s == m, flat, jnp.int32(2 ** 30))
        idx = jnp.min(cand)
        pos = flat == idx
        lv = jnp.max(jnp.where(pos, l_scr[...], 0))
        s_scr[...] = jnp.where(pos, -jnp.inf, s)
        scores_out[pl.ds(i, 1), :] = jnp.full((1, 1), m, jnp.float32)
        labels_out[pl.ds(i, 1), :] = jnp.full((1, 1), lv, jnp.int32)
        boxes_out[pl.ds(i, 1), :] = box_ref[pl.ds(idx, 1), :]
        return 0

    jax.lax.fori_loop(0, _K, body, 0)


def kernel(cls_preds, box_preds):
    clsT = jnp.transpose(cls_preds)
    clsp = jnp.pad(clsT, ((0, 0), (0, _R * _C - _N)))
    cls3 = clsp.reshape(_NCLS, _R, _C)
    boxp = jnp.pad(box_preds, ((0, _R * _C - _N), (0, 1)))

    ts, tb, tl = pl.pallas_call(
        _topk_body,
        out_shape=(
            jax.ShapeDtypeStruct((_KPAD, 1), jnp.float32),
            jax.ShapeDtypeStruct((_KPAD, 8), jnp.float32),
            jax.ShapeDtypeStruct((_KPAD, 1), jnp.int32),
        ),
        scratch_shapes=[
            pltpu.VMEM((_R, _C), jnp.float32),
            pltpu.VMEM((_R, _C), jnp.int32),
        ],
    )(cls3, boxp)

    return ts[:_K, 0], tb[:_K, :7], tl[:_K, 0]


# speedup vs baseline: 12.3191x; 12.3191x over previous
import jax
import jax.numpy as jnp
from jax.experimental import pallas as pl
from jax.experimental.pallas import tpu as pltpu

_N = 20000
_R, _C = 160, 128
_K = 500
_KPAD = 512
_CAND = 512
_NCLS = 10
_NBOX = 7
_NFIELDS = 3 + _NBOX


def _mm(a, b, ca, cb):
    return jax.lax.dot_general(
        a, b, (((ca,), (cb,)), ((), ())),
        precision=jax.lax.Precision.HIGHEST,
        preferred_element_type=jnp.float32,
    )


def _topk_body(cls_ref, box_ref, out_ref):
    x = cls_ref[...]
    best = x[0]
    lab = jnp.zeros((_R, _C), jnp.int32)
    for c in range(1, _NCLS):
        upd = x[c] > best
        best = jnp.where(upd, x[c], best)
        lab = jnp.where(upd, c, lab)

    ir = jax.lax.broadcasted_iota(jnp.int32, (_R, _C), 0)
    ic = jax.lax.broadcasted_iota(jnp.int32, (_R, _C), 1)
    flat = ir * _C + ic
    best = jnp.where(flat < _N, best, -jnp.inf)

    b = jax.lax.bitcast_convert_type(best, jnp.int32)
    key = jnp.where(b >= 0, b, b ^ jnp.int32(2 ** 31 - 1))

    kf = jnp.float32(_K)
    cnt_nn = jnp.sum(jnp.where(key >= 0, 1.0, 0.0))
    base = jnp.where(cnt_nn >= kf, jnp.int32(0), jnp.int32(-2 ** 31))

    def bit_body(t, p):
        bit = jnp.int32(1) << (jnp.int32(30) - t)
        candv = base + (p | bit)
        cnt = jnp.sum(jnp.where(key >= candv, 1.0, 0.0))
        return jnp.where(cnt >= kf, p | bit, p)

    p_low = jax.lax.fori_loop(0, 31, bit_body, jnp.int32(0))
    thr = base + p_low

    lane_lt = jnp.where(
        jax.lax.broadcasted_iota(jnp.int32, (_C, _C), 0)
        < jax.lax.broadcasted_iota(jnp.int32, (_C, _C), 1),
        1.0, 0.0)
    row_lt = jnp.where(
        jax.lax.broadcasted_iota(jnp.int32, (_R, _R), 1)
        < jax.lax.broadcasted_iota(jnp.int32, (_R, _R), 0),
        1.0, 0.0)

    def _flat_exclusive_cumsum(mask_f):
        lane_exc = _mm(mask_f, lane_lt, 1, 0)
        rowsum = jnp.sum(mask_f, axis=1, keepdims=True)
        rowcum_exc = _mm(row_lt, rowsum, 1, 0)
        return rowcum_exc + lane_exc, rowsum, rowcum_exc

    gt = key > thr
    cnt_gt = jnp.sum(jnp.where(gt, 1.0, 0.0))
    needed = kf - cnt_gt
    tie_f = jnp.where(key == thr, 1.0, 0.0)
    tie_exc, _, _ = _flat_exclusive_cumsum(tie_f)
    sel = gt | ((key == thr) & (tie_exc < needed))
    self_f = jnp.where(sel, 1.0, 0.0)
    slot, rowsum, rowcum_exc = _flat_exclusive_cumsum(self_f)

    iota_p_row = jax.lax.broadcasted_iota(jnp.int32, (_R, _CAND), 1).astype(jnp.float32)
    rt = jnp.where(
        (iota_p_row >= rowcum_exc) & (iota_p_row < rowcum_exc + rowsum),
        1.0, 0.0)

    slotm = jnp.where(sel, slot, -1.0)
    s_rows = _mm(rt, slotm, 0, 0)
    iota_pc = jax.lax.broadcasted_iota(jnp.int32, (_CAND, _C), 0).astype(jnp.float32)
    cmask = jnp.where(s_rows == iota_pc, 1.0, 0.0)

    best_fin = jnp.where(flat < _N, best, 0.0)
    payload = jnp.concatenate(
        [best_fin, flat.astype(jnp.float32), (lab + 1).astype(jnp.float32)]
        + [box_ref[j] for j in range(_NBOX)],
        axis=1)
    gath = _mm(rt, payload, 0, 0)
    cand = [
        jnp.sum(gath[:, f * _C:(f + 1) * _C] * cmask, axis=1, keepdims=True)
        for f in range(_NFIELDS)
    ]
    cscore, cidx = cand[0], cand[1]

    total = jnp.sum(self_f)
    iota_col = jax.lax.broadcasted_iota(jnp.int32, (_CAND, 1), 0).astype(jnp.float32)
    valid = iota_col < total
    cscore = jnp.where(valid, cscore, jnp.float32(-3.0e38))
    cidx = jnp.where(valid, cidx, jnp.float32(2 ** 25))

    eye = jnp.where(
        jax.lax.broadcasted_iota(jnp.int32, (_CAND, _CAND), 0)
        == jax.lax.broadcasted_iota(jnp.int32, (_CAND, _CAND), 1),
        1.0, 0.0)
    srow = _mm(cscore, eye, 0, 0)
    irow = _mm(cidx, eye, 0, 0)
    beats = jnp.where(srow > cscore, 1.0, 0.0) + jnp.where(
        (srow == cscore) & (irow < cidx), 1.0, 0.0)
    rank = jnp.sum(beats, axis=1, keepdims=True)

    iota_o = jax.lax.broadcasted_iota(jnp.int32, (_CAND, _KPAD), 1).astype(jnp.float32)
    pt = jnp.where(rank == iota_o, 1.0, 0.0)
    candmat = jnp.concatenate([cand[0], cand[2]] + cand[3:], axis=1)
    out_ref[...] = _mm(pt, candmat, 0, 0)


def kernel(cls_preds, box_preds):
    clsT = jnp.transpose(cls_preds)
    clsp = jnp.pad(clsT, ((0, 0), (0, _R * _C - _N)))
    cls3 = clsp.reshape(_NCLS, _R, _C)
    boxT = jnp.transpose(box_preds)
    boxp = jnp.pad(boxT, ((0, 0), (0, _R * _C - _N)))
    box3 = boxp.reshape(_NBOX, _R, _C)

    out = pl.pallas_call(
        _topk_body,
        out_shape=jax.ShapeDtypeStruct((_KPAD, 2 + _NBOX), jnp.float32),
    )(cls3, box3)

    top_scores = out[:_K, 0]
    top_labels = out[:_K, 1].astype(jnp.int32)
    top_boxes = out[:_K, 2:2 + _NBOX]
    return top_scores, top_boxes, top_labels


# speedup vs baseline: 12.6262x; 1.0249x over previous
import jax
import jax.numpy as jnp
from jax.experimental import pallas as pl
from jax.experimental.pallas import tpu as pltpu

_N = 20000
_R, _C = 160, 128
_K = 500
_KPAD = 512
_CAND = 512
_NCLS = 10
_NBOX = 7
_NFIELDS = 3 + _NBOX


def _mm(a, b, ca, cb):
    return jax.lax.dot_general(
        a, b, (((ca,), (cb,)), ((), ())),
        precision=jax.lax.Precision.HIGHEST,
        preferred_element_type=jnp.float32,
    )


def _topk_body(in_ref, out_ref):
    x = in_ref[...]
    best = x[0]
    lab = jnp.zeros((_R, _C), jnp.int32)
    for c in range(1, _NCLS):
        upd = x[c] > best
        best = jnp.where(upd, x[c], best)
        lab = jnp.where(upd, c, lab)

    ir = jax.lax.broadcasted_iota(jnp.int32, (_R, _C), 0)
    ic = jax.lax.broadcasted_iota(jnp.int32, (_R, _C), 1)
    flat = ir * _C + ic
    best = jnp.where(flat < _N, best, -jnp.inf)

    b = jax.lax.bitcast_convert_type(best, jnp.int32)
    key = jnp.where(b >= 0, b, b ^ jnp.int32(2 ** 31 - 1))

    kf = jnp.float32(_K)
    cnt_nn = jnp.sum(jnp.where(key >= 0, 1.0, 0.0))
    base = jnp.where(cnt_nn >= kf, jnp.int32(0), jnp.int32(-2 ** 31))

    def bit_body(t, p):
        bit = jnp.int32(1) << (jnp.int32(30) - t)
        candv = base + (p | bit)
        cnt = jnp.sum(jnp.where(key >= candv, 1.0, 0.0))
        return jnp.where(cnt >= kf, p | bit, p)

    p_low = jax.lax.fori_loop(0, 31, bit_body, jnp.int32(0))
    thr = base + p_low

    lane_lt = jnp.where(
        jax.lax.broadcasted_iota(jnp.int32, (_C, _C), 0)
        < jax.lax.broadcasted_iota(jnp.int32, (_C, _C), 1),
        1.0, 0.0)
    row_lt = jnp.where(
        jax.lax.broadcasted_iota(jnp.int32, (_R, _R), 1)
        < jax.lax.broadcasted_iota(jnp.int32, (_R, _R), 0),
        1.0, 0.0)

    def _flat_exclusive_cumsum(mask_f):
        lane_exc = _mm(mask_f, lane_lt, 1, 0)
        rowsum = jnp.sum(mask_f, axis=1, keepdims=True)
        rowcum_exc = _mm(row_lt, rowsum, 1, 0)
        return rowcum_exc + lane_exc, rowsum, rowcum_exc

    gt = key > thr
    cnt_gt = jnp.sum(jnp.where(gt, 1.0, 0.0))
    needed = kf - cnt_gt
    tie_f = jnp.where(key == thr, 1.0, 0.0)
    tie_exc, _, _ = _flat_exclusive_cumsum(tie_f)
    sel = gt | ((key == thr) & (tie_exc < needed))
    self_f = jnp.where(sel, 1.0, 0.0)
    slot, rowsum, rowcum_exc = _flat_exclusive_cumsum(self_f)

    iota_p_row = jax.lax.broadcasted_iota(jnp.int32, (_R, _CAND), 1).astype(jnp.float32)
    rt = jnp.where(
        (iota_p_row >= rowcum_exc) & (iota_p_row < rowcum_exc + rowsum),
        1.0, 0.0)

    slotm = jnp.where(sel, slot, -1.0)
    s_rows = _mm(rt, slotm, 0, 0)
    iota_pc = jax.lax.broadcasted_iota(jnp.int32, (_CAND, _C), 0).astype(jnp.float32)
    cmask = jnp.where(s_rows == iota_pc, 1.0, 0.0)

    best_fin = jnp.where(flat < _N, best, 0.0)
    payload = jnp.concatenate(
        [best_fin, flat.astype(jnp.float32), (lab + 1).astype(jnp.float32)]
        + [x[_NCLS + j] for j in range(_NBOX)],
        axis=1)
    gath = _mm(rt, payload, 0, 0)
    cand = [
        jnp.sum(gath[:, f * _C:(f + 1) * _C] * cmask, axis=1, keepdims=True)
        for f in range(_NFIELDS)
    ]
    cscore, cidx = cand[0], cand[1]

    total = jnp.sum(self_f)
    iota_col = jax.lax.broadcasted_iota(jnp.int32, (_CAND, 1), 0).astype(jnp.float32)
    valid = iota_col < total
    cscore = jnp.where(valid, cscore, jnp.float32(-3.0e38))
    cidx = jnp.where(valid, cidx, jnp.float32(2 ** 25))

    eye = jnp.where(
        jax.lax.broadcasted_iota(jnp.int32, (_CAND, _CAND), 0)
        == jax.lax.broadcasted_iota(jnp.int32, (_CAND, _CAND), 1),
        1.0, 0.0)
    srow = _mm(cscore, eye, 0, 0)
    irow = _mm(cidx, eye, 0, 0)
    beats = jnp.where(srow > cscore, 1.0, 0.0) + jnp.where(
        (srow == cscore) & (irow < cidx), 1.0, 0.0)
    rank = jnp.sum(beats, axis=1, keepdims=True)

    iota_o = jax.lax.broadcasted_iota(jnp.int32, (_CAND, _KPAD), 1).astype(jnp.float32)
    pt = jnp.where(rank == iota_o, 1.0, 0.0)
    candmat = jnp.concatenate([cand[0], cand[2]] + cand[3:], axis=1)
    out_ref[...] = _mm(pt, candmat, 0, 0)


def kernel(cls_preds, box_preds):
    allT = jnp.transpose(jnp.concatenate([cls_preds, box_preds], axis=1))
    allp = jnp.pad(allT, ((0, 0), (0, _R * _C - _N)))
    all3 = allp.reshape(_NCLS + _NBOX, _R, _C)

    out = pl.pallas_call(
        _topk_body,
        out_shape=jax.ShapeDtypeStruct((_KPAD, 2 + _NBOX), jnp.float32),
    )(all3)

    top_scores = out[:_K, 0]
    top_labels = out[:_K, 1].astype(jnp.int32)
    top_boxes = out[:_K, 2:2 + _NBOX]
    return top_scores, top_boxes, top_labels
